# baseline (speedup 1.0000x reference)
import jax
import jax.numpy as jnp
from jax import lax
from jax.experimental import pallas as pl
from jax.experimental.pallas import tpu as pltpu
from jax.experimental.pallas import tpu_sc as plsc

N_NODES = 10000
N_EDGES = 320000
D = 128

NC = 2
NS = 16
L = 16

CHUNK = 112
NCH = 90
EPW = NCH * CHUNK
E_PAD = NC * NS * EPW

OB = 40
NB = N_NODES // OB
BPS = -(-NB // NS)


def _spmm_body(x_hbm, idx_hbm, w_hbm, part_hbm,
               idxs, ws, rb0, rb1, sb0, sb1, db0, db1,
               g0, g1, s0, s1, acc):
    c = lax.axis_index("c")
    s = lax.axis_index("s")
    rows = (rb0, rb1)
    sbuf = (sb0, sb1)
    dbuf = (db0, db1)
    gsem = (g0, g1)
    ssem = (s0, s1)

    base = (c * NS + s) * EPW
    cp_idx = pltpu.async_copy(idx_hbm.at[pl.ds(base, EPW)], idxs, g0)
    cp_w = pltpu.async_copy(w_hbm.at[pl.ds(base, EPW)], ws, g1)

    zero = jnp.zeros((L,), jnp.float32)

    def zfill(i, _):
        for k in range(D // L):
            rb0[i, pl.ds(k * L, L)] = zero
        return 0

    lax.fori_loop(0, OB, zfill, 0)
    for k in range(BPS):
        b = s + k * NS

        @pl.when(b < NB)
        def _():
            pltpu.sync_copy(rb0.at[pl.ds(0, OB)], acc.at[pl.ds(b * OB, OB)])

    cp_idx.wait()
    cp_w.wait()
    plsc.subcore_barrier()

    def decode(j, b):
        for g in range(CHUNK // L):
            v = idxs[pl.ds(j * CHUNK + g * L, L)]
            sl = pl.ds(g * L, L)
            sbuf[b][sl] = v & 0xFFFF
            dbuf[b][sl] = lax.shift_right_logical(v, 16)

    def start_gather(j, b):
        decode(j, b)
        pltpu.async_copy(x_hbm.at[sbuf[b]], rows[b], gsem[b])

    def wait_gather(b):
        pltpu.make_async_copy(x_hbm.at[sbuf[b]], rows[b], gsem[b]).wait()

    def start_scatter(b):
        pltpu.async_copy(rows[b], acc.at[dbuf[b]], ssem[b], add=True)

    def wait_scatter(b):
        pltpu.make_async_copy(rows[b], acc.at[dbuf[b]], ssem[b]).wait()

    def scale(j, b):
        rb = rows[b]

        def mul(g, _):
            wv = ws[pl.ds(j * CHUNK + g * L, L)]
            for t in range(L):
                w = wv[t]
                i = g * L + t
                for k in range(D // L):
                    sl = pl.ds(k * L, L)
                    rb[i, sl] = rb[i, sl] * w
            return 0

        lax.fori_loop(0, CHUNK // L, mul, 0)

    start_gather(0, 0)
    start_gather(1, 1)
    wait_gather(0)
    scale(0, 0)
    start_scatter(0)

    def outer(k2, _):
        for u in range(2):
            j = 2 * k2 + 1 + u
            b = (1 + u) % 2
            nb = u % 2
            wait_scatter(nb)
            start_gather(j + 1, nb)
            wait_gather(b)
            scale(j, b)
            start_scatter(b)
        return 0

    lax.fori_loop(0, (NCH - 2) // 2, outer, 0)

    wait_gather((NCH - 1) % 2)
    scale(NCH - 1, (NCH - 1) % 2)
    start_scatter((NCH - 1) % 2)
    wait_scatter((NCH - 2) % 2)
    wait_scatter((NCH - 1) % 2)
    plsc.subcore_barrier()

    for k in range(BPS):
        b = s + k * NS

        @pl.when(b < NB)
        def _():
            r = b * OB
            pltpu.sync_copy(acc.at[pl.ds(r, OB)], rb0.at[pl.ds(0, OB)])
            pltpu.sync_copy(rb0.at[pl.ds(0, OB)], part_hbm.at[c, pl.ds(r, OB)])


_spmm = pl.kernel(
    _spmm_body,
    out_type=jax.ShapeDtypeStruct((NC, N_NODES, D), jnp.float32),
    mesh=plsc.VectorSubcoreMesh(core_axis_name="c", subcore_axis_name="s",
                                num_cores=NC, num_subcores=NS),
    scratch_types=[
        pltpu.VMEM((EPW,), jnp.int32),
        pltpu.VMEM((EPW,), jnp.float32),
        pltpu.VMEM((CHUNK, D), jnp.float32),
        pltpu.VMEM((CHUNK, D), jnp.float32),
        pltpu.VMEM((CHUNK,), jnp.int32),
        pltpu.VMEM((CHUNK,), jnp.int32),
        pltpu.VMEM((CHUNK,), jnp.int32),
        pltpu.VMEM((CHUNK,), jnp.int32),
        pltpu.SemaphoreType.DMA,
        pltpu.SemaphoreType.DMA,
        pltpu.SemaphoreType.DMA,
        pltpu.SemaphoreType.DMA,
        pltpu.VMEM_SHARED((N_NODES, D), jnp.float32),
    ],
)


def _mm_body(x_ref, w_ref, o_ref):
    o_ref[...] = jnp.dot(x_ref[...], w_ref[...],
                         preferred_element_type=jnp.float32)


def _fuse_body(p_ref, w_ref, o_ref):
    h = jnp.maximum(p_ref[0] + p_ref[1], 0.0)
    o_ref[...] = jnp.dot(h, w_ref[...], preferred_element_type=jnp.float32)


def _add_body(q_ref, o_ref):
    o_ref[...] = q_ref[0] + q_ref[1]


_MB = 1000

_mm = pl.pallas_call(
    _mm_body,
    grid=(N_NODES // _MB,),
    in_specs=[pl.BlockSpec((_MB, D), lambda i: (i, 0)),
              pl.BlockSpec((D, D), lambda i: (0, 0))],
    out_specs=pl.BlockSpec((_MB, D), lambda i: (i, 0)),
    out_shape=jax.ShapeDtypeStruct((N_NODES, D), jnp.float32),
)

_fuse = pl.pallas_call(
    _fuse_body,
    grid=(N_NODES // _MB,),
    in_specs=[pl.BlockSpec((NC, _MB, D), lambda i: (0, i, 0)),
              pl.BlockSpec((D, D), lambda i: (0, 0))],
    out_specs=pl.BlockSpec((_MB, D), lambda i: (i, 0)),
    out_shape=jax.ShapeDtypeStruct((N_NODES, D), jnp.float32),
)

_add = pl.pallas_call(
    _add_body,
    grid=(N_NODES // _MB,),
    in_specs=[pl.BlockSpec((NC, _MB, D), lambda i: (0, i, 0))],
    out_specs=pl.BlockSpec((_MB, D), lambda i: (i, 0)),
    out_shape=jax.ShapeDtypeStruct((N_NODES, D), jnp.float32),
)


@jax.jit
def kernel(features, edge_index, edge_weight, W1, W2):
    pad = E_PAD - N_EDGES
    src = edge_index[0].astype(jnp.int32)
    dst = edge_index[1].astype(jnp.int32)
    pad_dst = jnp.arange(pad, dtype=jnp.int32) % N_NODES
    packed = jnp.concatenate(
        [(dst << 16) | src, pad_dst << 16]
    )
    w = jnp.concatenate(
        [edge_weight.astype(jnp.float32), jnp.zeros((pad,), jnp.float32)]
    )

    s1 = _mm(features, W1)
    p = _spmm(s1, packed, w)
    s2 = _fuse(p, W2)
    q = _spmm(s2, packed, w)
    return _add(q)


# speedup vs baseline: 1.0354x; 1.0354x over previous
import jax
import jax.numpy as jnp
from jax import lax
from jax.experimental import pallas as pl
from jax.experimental.pallas import tpu as pltpu
from jax.experimental.pallas import tpu_sc as plsc

N_NODES = 10000
N_EDGES = 320000
D = 128

NC = 2
NS = 16
L = 16

CHUNK = 112
NCH = 90
EPW = NCH * CHUNK
E_PAD = NC * NS * EPW

SPAN = 624
REM = N_NODES - NS * SPAN


def _spmm_body(x_hbm, idx_hbm, w_hbm, part_hbm,
               idxs, ws, rb0, rb1, sb0, sb1, db0, db1,
               g0, g1, s0, s1, acc):
    c = lax.axis_index("c")
    s = lax.axis_index("s")
    rows = (rb0, rb1)
    sbuf = (sb0, sb1)
    dbuf = (db0, db1)
    gsem = (g0, g1)
    ssem = (s0, s1)

    base = (c * NS + s) * EPW
    cp_idx = pltpu.async_copy(idx_hbm.at[pl.ds(base, EPW)], idxs, g0)
    cp_w = pltpu.async_copy(w_hbm.at[pl.ds(base, EPW)], ws, g1)

    zero = jnp.zeros((L,), jnp.float32)
    zrow = s * SPAN
    ZF = SPAN // CHUNK
    ZR = SPAN % CHUNK

    def zfill(i, _):
        for k in range(D // L):
            rb0[i, pl.ds(k * L, L)] = zero
        return 0

    lax.fori_loop(0, CHUNK, zfill, 0)
    for k in range(ZF):
        pltpu.async_copy(rb0, acc.at[pl.ds(zrow + k * CHUNK, CHUNK)], s0)
    pltpu.async_copy(rb0.at[pl.ds(0, ZR)],
                     acc.at[pl.ds(zrow + ZF * CHUNK, ZR)], s1)

    @pl.when(s == 0)
    def _():
        pltpu.async_copy(rb0.at[pl.ds(0, REM)],
                         acc.at[pl.ds(NS * SPAN, REM)], s1)

    for k in range(ZF):
        pltpu.make_async_copy(rb0, acc.at[pl.ds(zrow + k * CHUNK, CHUNK)],
                              s0).wait()
    pltpu.make_async_copy(rb0.at[pl.ds(0, ZR)],
                          acc.at[pl.ds(zrow + ZF * CHUNK, ZR)], s1).wait()

    @pl.when(s == 0)
    def _():
        pltpu.make_async_copy(rb0.at[pl.ds(0, REM)],
                              acc.at[pl.ds(NS * SPAN, REM)], s1).wait()

    cp_idx.wait()
    cp_w.wait()
    plsc.subcore_barrier()

    def decode(j, b):
        for g in range(CHUNK // L):
            v = idxs[pl.ds(j * CHUNK + g * L, L)]
            sl = pl.ds(g * L, L)
            sbuf[b][sl] = v & 0xFFFF
            dbuf[b][sl] = lax.shift_right_logical(v, 16)

    def start_gather(j, b):
        decode(j, b)
        pltpu.async_copy(x_hbm.at[sbuf[b]], rows[b], gsem[b])

    def wait_gather(b):
        pltpu.make_async_copy(x_hbm.at[sbuf[b]], rows[b], gsem[b]).wait()

    def start_scatter(b):
        pltpu.async_copy(rows[b], acc.at[dbuf[b]], ssem[b], add=True)

    def wait_scatter(b):
        pltpu.make_async_copy(rows[b], acc.at[dbuf[b]], ssem[b]).wait()

    def scale(j, b):
        rb = rows[b]

        def mul(g, _):
            wv = ws[pl.ds(j * CHUNK + g * L, L)]
            for t in range(L):
                w = wv[t]
                i = g * L + t
                for k in range(D // L):
                    sl = pl.ds(k * L, L)
                    rb[i, sl] = rb[i, sl] * w
            return 0

        lax.fori_loop(0, CHUNK // L, mul, 0)

    start_gather(0, 0)
    start_gather(1, 1)
    wait_gather(0)
    scale(0, 0)
    start_scatter(0)

    def outer(k2, _):
        for u in range(2):
            j = 2 * k2 + 1 + u
            b = (1 + u) % 2
            nb = u % 2
            wait_scatter(nb)
            start_gather(j + 1, nb)
            wait_gather(b)
            scale(j, b)
            start_scatter(b)
        return 0

    lax.fori_loop(0, (NCH - 2) // 2, outer, 0)

    wait_gather((NCH - 1) % 2)
    scale(NCH - 1, (NCH - 1) % 2)
    start_scatter((NCH - 1) % 2)
    wait_scatter((NCH - 2) % 2)
    wait_scatter((NCH - 1) % 2)
    plsc.subcore_barrier()

    pltpu.async_copy(acc.at[pl.ds(zrow, SPAN)],
                     part_hbm.at[c, pl.ds(zrow, SPAN)], g0)

    @pl.when(s == 0)
    def _():
        pltpu.async_copy(acc.at[pl.ds(NS * SPAN, REM)],
                         part_hbm.at[c, pl.ds(NS * SPAN, REM)], g1)
        pltpu.make_async_copy(acc.at[pl.ds(NS * SPAN, REM)],
                              part_hbm.at[c, pl.ds(NS * SPAN, REM)],
                              g1).wait()

    pltpu.make_async_copy(acc.at[pl.ds(zrow, SPAN)],
                          part_hbm.at[c, pl.ds(zrow, SPAN)], g0).wait()


_spmm = pl.kernel(
    _spmm_body,
    out_type=jax.ShapeDtypeStruct((NC, N_NODES, D), jnp.float32),
    mesh=plsc.VectorSubcoreMesh(core_axis_name="c", subcore_axis_name="s",
                                num_cores=NC, num_subcores=NS),
    scratch_types=[
        pltpu.VMEM((EPW,), jnp.int32),
        pltpu.VMEM((EPW,), jnp.float32),
        pltpu.VMEM((CHUNK, D), jnp.float32),
        pltpu.VMEM((CHUNK, D), jnp.float32),
        pltpu.VMEM((CHUNK,), jnp.int32),
        pltpu.VMEM((CHUNK,), jnp.int32),
        pltpu.VMEM((CHUNK,), jnp.int32),
        pltpu.VMEM((CHUNK,), jnp.int32),
        pltpu.SemaphoreType.DMA,
        pltpu.SemaphoreType.DMA,
        pltpu.SemaphoreType.DMA,
        pltpu.SemaphoreType.DMA,
        pltpu.VMEM_SHARED((N_NODES, D), jnp.float32),
    ],
)


def _mm_body(x_ref, w_ref, o_ref):
    o_ref[...] = jnp.dot(x_ref[...], w_ref[...],
                         preferred_element_type=jnp.float32)


def _fuse_body(p_ref, w_ref, o_ref):
    h = jnp.maximum(p_ref[0] + p_ref[1], 0.0)
    o_ref[...] = jnp.dot(h, w_ref[...], preferred_element_type=jnp.float32)


def _add_body(q_ref, o_ref):
    o_ref[...] = q_ref[0] + q_ref[1]


_MB = 1000

_mm = pl.pallas_call(
    _mm_body,
    grid=(N_NODES // _MB,),
    in_specs=[pl.BlockSpec((_MB, D), lambda i: (i, 0)),
              pl.BlockSpec((D, D), lambda i: (0, 0))],
    out_specs=pl.BlockSpec((_MB, D), lambda i: (i, 0)),
    out_shape=jax.ShapeDtypeStruct((N_NODES, D), jnp.float32),
)

_fuse = pl.pallas_call(
    _fuse_body,
    grid=(N_NODES // _MB,),
    in_specs=[pl.BlockSpec((NC, _MB, D), lambda i: (0, i, 0)),
              pl.BlockSpec((D, D), lambda i: (0, 0))],
    out_specs=pl.BlockSpec((_MB, D), lambda i: (i, 0)),
    out_shape=jax.ShapeDtypeStruct((N_NODES, D), jnp.float32),
)

_add = pl.pallas_call(
    _add_body,
    grid=(N_NODES // _MB,),
    in_specs=[pl.BlockSpec((NC, _MB, D), lambda i: (0, i, 0))],
    out_specs=pl.BlockSpec((_MB, D), lambda i: (i, 0)),
    out_shape=jax.ShapeDtypeStruct((N_NODES, D), jnp.float32),
)


@jax.jit
def kernel(features, edge_index, edge_weight, W1, W2):
    pad = E_PAD - N_EDGES
    src = edge_index[0].astype(jnp.int32)
    dst = edge_index[1].astype(jnp.int32)
    pad_dst = jnp.arange(pad, dtype=jnp.int32) % N_NODES
    packed = jnp.concatenate(
        [(dst << 16) | src, pad_dst << 16]
    )
    w = jnp.concatenate(
        [edge_weight.astype(jnp.float32), jnp.zeros((pad,), jnp.float32)]
    )

    s1 = _mm(features, W1)
    p = _spmm(s1, packed, w)
    s2 = _fuse(p, W2)
    q = _spmm(s2, packed, w)
    return _add(q)


# speedup vs baseline: 1.7327x; 1.6734x over previous
import jax
import jax.numpy as jnp
from jax import lax
from jax.experimental import pallas as pl
from jax.experimental.pallas import tpu as pltpu
from jax.experimental.pallas import tpu_sc as plsc

N_NODES = 10000
N_EDGES = 320000
D = 128

NC = 2
NS = 16
L = 16

CHUNK = 112
NCH = 90
EPW = NCH * CHUNK
E_PAD = NC * NS * EPW

SPAN = 624
REM = N_NODES - NS * SPAN


def _spmm_body(x_hbm, idx_hbm, w_hbm, part_hbm,
               idxs, ws, rb0, rb1, sb0, sb1, db0, db1,
               g0, g1, s0, s1, acc):
    c = lax.axis_index("c")
    s = lax.axis_index("s")
    rows = (rb0, rb1)
    sbuf = (sb0, sb1)
    dbuf = (db0, db1)
    gsem = (g0, g1)
    ssem = (s0, s1)

    base = (c * NS + s) * EPW
    cp_idx = pltpu.async_copy(idx_hbm.at[pl.ds(base, EPW)], idxs, g0)
    cp_w = pltpu.async_copy(w_hbm.at[pl.ds(base, EPW)], ws, g1)

    zero = jnp.zeros((L,), jnp.float32)
    zrow = s * SPAN
    ZF = SPAN // CHUNK
    ZR = SPAN % CHUNK

    def zfill(i, _):
        for k in range(D // L):
            rb0[i, pl.ds(k * L, L)] = zero
        return 0

    lax.fori_loop(0, CHUNK, zfill, 0)
    for k in range(ZF):
        pltpu.async_copy(rb0, acc.at[pl.ds(zrow + k * CHUNK, CHUNK)], s0)
    pltpu.async_copy(rb0.at[pl.ds(0, ZR)],
                     acc.at[pl.ds(zrow + ZF * CHUNK, ZR)], s1)

    @pl.when(s == 0)
    def _():
        pltpu.async_copy(rb0.at[pl.ds(0, REM)],
                         acc.at[pl.ds(NS * SPAN, REM)], s1)

    for k in range(ZF):
        pltpu.make_async_copy(rb0, acc.at[pl.ds(zrow + k * CHUNK, CHUNK)],
                              s0).wait()
    pltpu.make_async_copy(rb0.at[pl.ds(0, ZR)],
                          acc.at[pl.ds(zrow + ZF * CHUNK, ZR)], s1).wait()

    @pl.when(s == 0)
    def _():
        pltpu.make_async_copy(rb0.at[pl.ds(0, REM)],
                              acc.at[pl.ds(NS * SPAN, REM)], s1).wait()

    cp_idx.wait()
    cp_w.wait()
    plsc.subcore_barrier()

    def decode(j, b):
        for g in range(CHUNK // L):
            v = idxs[pl.ds(j * CHUNK + g * L, L)]
            sl = pl.ds(g * L, L)
            sbuf[b][sl] = v & 0xFFFF
            dbuf[b][sl] = lax.shift_right_logical(v, 16)

    def start_gather(j, b):
        decode(j, b)
        pltpu.async_copy(x_hbm.at[sbuf[b]], rows[b], gsem[b])

    def wait_gather(b):
        pltpu.make_async_copy(x_hbm.at[sbuf[b]], rows[b], gsem[b]).wait()

    def start_scatter(b):
        pltpu.async_copy(rows[b], acc.at[dbuf[b]], ssem[b], add=True)

    def wait_scatter(b):
        pltpu.make_async_copy(rows[b], acc.at[dbuf[b]], ssem[b]).wait()

    def scale(j, b):
        rb = rows[b]

        def mul(g, _):
            wv = ws[pl.ds(j * CHUNK + g * L, L)]
            for t in range(L):
                w = wv[t]
                i = g * L + t
                for k in range(D // L):
                    sl = pl.ds(k * L, L)
                    rb[i, sl] = rb[i, sl] * w
            return 0

        lax.fori_loop(0, CHUNK // L, mul, 0)

    start_gather(0, 0)
    start_gather(1, 1)
    wait_gather(0)
    scale(0, 0)
    start_scatter(0)

    def outer(k2, _):
        for u in range(2):
            j = 2 * k2 + 1 + u
            b = (1 + u) % 2
            nb = u % 2
            wait_scatter(nb)
            start_gather(j + 1, nb)
            wait_gather(b)
            scale(j, b)
            start_scatter(b)
        return 0

    lax.fori_loop(0, (NCH - 2) // 2, outer, 0)

    wait_gather((NCH - 1) % 2)
    scale(NCH - 1, (NCH - 1) % 2)
    start_scatter((NCH - 1) % 2)
    wait_scatter((NCH - 2) % 2)
    wait_scatter((NCH - 1) % 2)
    plsc.subcore_barrier()

    pltpu.async_copy(acc.at[pl.ds(zrow, SPAN)],
                     part_hbm.at[c, pl.ds(zrow, SPAN)], g0)

    @pl.when(s == 0)
    def _():
        pltpu.async_copy(acc.at[pl.ds(NS * SPAN, REM)],
                         part_hbm.at[c, pl.ds(NS * SPAN, REM)], g1)
        pltpu.make_async_copy(acc.at[pl.ds(NS * SPAN, REM)],
                              part_hbm.at[c, pl.ds(NS * SPAN, REM)],
                              g1).wait()

    pltpu.make_async_copy(acc.at[pl.ds(zrow, SPAN)],
                          part_hbm.at[c, pl.ds(zrow, SPAN)], g0).wait()


_spmm = pl.kernel(
    _spmm_body,
    out_type=jax.ShapeDtypeStruct((NC, N_NODES, D), jnp.float32),
    mesh=plsc.VectorSubcoreMesh(core_axis_name="c", subcore_axis_name="s",
                                num_cores=NC, num_subcores=NS),
    scratch_types=[
        pltpu.VMEM((EPW,), jnp.int32),
        pltpu.VMEM((EPW,), jnp.float32),
        pltpu.VMEM((CHUNK, D), jnp.float32),
        pltpu.VMEM((CHUNK, D), jnp.float32),
        pltpu.VMEM((CHUNK,), jnp.int32),
        pltpu.VMEM((CHUNK,), jnp.int32),
        pltpu.VMEM((CHUNK,), jnp.int32),
        pltpu.VMEM((CHUNK,), jnp.int32),
        pltpu.SemaphoreType.DMA,
        pltpu.SemaphoreType.DMA,
        pltpu.SemaphoreType.DMA,
        pltpu.SemaphoreType.DMA,
        pltpu.VMEM_SHARED((N_NODES, D), jnp.float32),
    ],
)


def _mm_body(x_ref, w_ref, o_ref):
    o_ref[...] = jnp.dot(x_ref[...], w_ref[...],
                         preferred_element_type=jnp.float32)


def _fuse_body(p_ref, w_ref, o_ref):
    h = jnp.maximum(p_ref[0] + p_ref[1], 0.0)
    o_ref[...] = jnp.dot(h, w_ref[...], preferred_element_type=jnp.float32)


def _add_body(q_ref, o_ref):
    o_ref[...] = q_ref[0] + q_ref[1]


_MB = 1000

_mm = pl.pallas_call(
    _mm_body,
    grid=(N_NODES // _MB,),
    in_specs=[pl.BlockSpec((_MB, D), lambda i: (i, 0)),
              pl.BlockSpec((D, D), lambda i: (0, 0))],
    out_specs=pl.BlockSpec((_MB, D), lambda i: (i, 0)),
    out_shape=jax.ShapeDtypeStruct((N_NODES, D), jnp.float32),
)

_fuse = pl.pallas_call(
    _fuse_body,
    grid=(N_NODES // _MB,),
    in_specs=[pl.BlockSpec((NC, _MB, D), lambda i: (0, i, 0)),
              pl.BlockSpec((D, D), lambda i: (0, 0))],
    out_specs=pl.BlockSpec((_MB, D), lambda i: (i, 0)),
    out_shape=jax.ShapeDtypeStruct((N_NODES, D), jnp.float32),
)

_add = pl.pallas_call(
    _add_body,
    grid=(N_NODES // _MB,),
    in_specs=[pl.BlockSpec((NC, _MB, D), lambda i: (0, i, 0))],
    out_specs=pl.BlockSpec((_MB, D), lambda i: (i, 0)),
    out_shape=jax.ShapeDtypeStruct((N_NODES, D), jnp.float32),
)


@jax.jit
def kernel(features, edge_index, edge_weight, W1, W2):
    pad = E_PAD - N_EDGES
    src = edge_index[0].astype(jnp.int32)
    dst = edge_index[1].astype(jnp.int32)
    pad_ix = jnp.arange(pad, dtype=jnp.int32) % N_NODES
    packed = jnp.concatenate(
        [(dst << 16) | src, (pad_ix << 16) | pad_ix]
    )
    w = jnp.concatenate(
        [edge_weight.astype(jnp.float32), jnp.zeros((pad,), jnp.float32)]
    )

    s1 = _mm(features, W1)
    p = _spmm(s1, packed, w)
    s2 = _fuse(p, W2)
    q = _spmm(s2, packed, w)
    return _add(q)
